# baseline (speedup 1.0000x reference)
import jax
import jax.numpy as jnp
from jax import lax
from jax.experimental import pallas as pl
from jax.experimental.pallas import tpu as pltpu

INPUT_DIM = 768
NUM_EXPERTS = 64
BLOCK_COLS = 1024


def _router_kernel(x_ref, w_ref, b_ref, gw_ref, idx_ref):
    B, C, D = x_ref.shape
    x = x_ref[...].reshape(B * C, D)
    logits = lax.dot_general(
        x, w_ref[...],
        dimension_numbers=(((1,), (1,)), ((), ())),
        preferred_element_type=jnp.float32,
    ) + b_ref[...]
    m = jnp.max(logits, axis=-1, keepdims=True)
    e = jnp.exp(logits - m)
    w = e / jnp.sum(e, axis=-1, keepdims=True)
    gw_ref[...] = w.reshape(B, C, NUM_EXPERTS)
    onehot = jnp.where(logits == m, 1.0, 0.0).astype(jnp.float32)
    iexp = lax.broadcasted_iota(jnp.int32, (NUM_EXPERTS, 1), 0)
    p2 = lax.bitcast_convert_type((127 - iexp) << 23, jnp.float32)
    s2 = lax.dot_general(
        onehot, p2,
        dimension_numbers=(((1,), (0,)), ((), ())),
        preferred_element_type=jnp.float32,
    )
    first_max = 127 - lax.shift_right_logical(
        lax.bitcast_convert_type(s2, jnp.int32), 23
    )
    idx_ref[...] = first_max.reshape(B, C)


def kernel(x, W, b):
    B, S, D = x.shape
    nblk = S // BLOCK_COLS
    gw, idx = pl.pallas_call(
        _router_kernel,
        grid=(nblk,),
        in_specs=[
            pl.BlockSpec((B, BLOCK_COLS, D), lambda i: (0, i, 0)),
            pl.BlockSpec((NUM_EXPERTS, D), lambda i: (0, 0)),
            pl.BlockSpec((1, NUM_EXPERTS), lambda i: (0, 0)),
        ],
        out_specs=[
            pl.BlockSpec((B, BLOCK_COLS, NUM_EXPERTS), lambda i: (0, i, 0)),
            pl.BlockSpec((B, BLOCK_COLS), lambda i: (0, i)),
        ],
        out_shape=[
            jax.ShapeDtypeStruct((B, S, NUM_EXPERTS), jnp.float32),
            jax.ShapeDtypeStruct((B, S), jnp.int32),
        ],
        compiler_params=pltpu.CompilerParams(
            dimension_semantics=("parallel",),
        ),
    )(x, W, b.reshape(1, NUM_EXPERTS))
    return gw, idx


# speedup vs baseline: 1.5504x; 1.5504x over previous
import jax
import jax.numpy as jnp
from jax import lax
from jax.experimental import pallas as pl
from jax.experimental.pallas import tpu as pltpu

INPUT_DIM = 768
NUM_EXPERTS = 64
BLOCK_COLS = 1024


def _router_kernel(x_ref, w_ref, b_ref, gw_ref, idx_ref):
    B, C, D = x_ref.shape
    x = x_ref[...].reshape(B * C, D)
    logits = lax.dot_general(
        x, w_ref[...],
        dimension_numbers=(((1,), (1,)), ((), ())),
        preferred_element_type=jnp.float32,
    ) + b_ref[...]
    m = jnp.max(logits, axis=-1, keepdims=True)
    e = jnp.exp(logits - m)
    w = e / jnp.sum(e, axis=-1, keepdims=True)
    gw_ref[...] = w.reshape(B, C, NUM_EXPERTS)
    ids = lax.broadcasted_iota(jnp.int32, logits.shape, 1)
    first_max = jnp.min(
        jnp.where(logits == m, ids, NUM_EXPERTS), axis=-1
    )
    idx_ref[...] = first_max.reshape(B, C)


def kernel(x, W, b):
    B, S, D = x.shape
    nblk = S // BLOCK_COLS
    gw, idx = pl.pallas_call(
        _router_kernel,
        grid=(nblk,),
        in_specs=[
            pl.BlockSpec((B, BLOCK_COLS, D), lambda i: (0, i, 0)),
            pl.BlockSpec((NUM_EXPERTS, D), lambda i: (0, 0)),
            pl.BlockSpec((1, NUM_EXPERTS), lambda i: (0, 0)),
        ],
        out_specs=[
            pl.BlockSpec((B, BLOCK_COLS, NUM_EXPERTS), lambda i: (0, i, 0)),
            pl.BlockSpec((B, BLOCK_COLS), lambda i: (0, i)),
        ],
        out_shape=[
            jax.ShapeDtypeStruct((B, S, NUM_EXPERTS), jnp.float32),
            jax.ShapeDtypeStruct((B, S), jnp.int32),
        ],
        compiler_params=pltpu.CompilerParams(
            dimension_semantics=("parallel",),
        ),
    )(x, W, b.reshape(1, NUM_EXPERTS))
    return gw, idx
